# baseline (device time: 27920 ns/iter reference)
import jax
import jax.numpy as jnp
from jax import lax
from jax.experimental import pallas as pl
from jax.experimental.pallas import tpu as pltpu

NCHUNK = 4


def kernel(x, assign, W1, W2):
    t, d = x.shape
    n_exp, _, f = W1.shape
    tc = t // NCHUNK
    assign2d = assign.reshape(t, 1)
    xb = x.astype(jnp.bfloat16)
    w1b = W1.astype(jnp.bfloat16)
    w2b = W2.astype(jnp.bfloat16)

    def body(x_ref, a_ref, w1_ref, w2_ref, out_ref,
             xr_ref, ar_ref, ys_ref, yr_ref,
             x_send_sems, x_recv_sems, a_sems, y_send_sems, y_recv_sems):
        my_x = lax.axis_index("x")
        my_y = lax.axis_index("y")
        my_z = lax.axis_index("z")
        peer = (my_x, my_y, 1 - my_z)

        barrier = pltpu.get_barrier_semaphore()
        pl.semaphore_signal(barrier, inc=1, device_id=peer,
                            device_id_type=pl.DeviceIdType.MESH)
        pl.semaphore_wait(barrier, 1)

        chunk = lambda ref, i: ref.at[pl.ds(i * tc, tc), :]
        x_rdmas = []
        for i in range(NCHUNK):
            r = pltpu.make_async_remote_copy(
                src_ref=chunk(x_ref, i), dst_ref=chunk(xr_ref, i),
                send_sem=x_send_sems.at[i], recv_sem=x_recv_sems.at[i],
                device_id=peer, device_id_type=pl.DeviceIdType.MESH)
            r.start()
            x_rdmas.append(r)
        rdma_a = pltpu.make_async_remote_copy(
            src_ref=a_ref, dst_ref=ar_ref,
            send_sem=a_sems.at[0], recv_sem=a_sems.at[1],
            device_id=peer, device_id_type=pl.DeviceIdType.MESH)
        rdma_a.start()

        e_base = 2 * my_z

        def ffn(x_blk, a_blk):
            m = x_blk.shape[0]
            acc = jnp.zeros((m, d), jnp.float32)
            for el in range(n_exp):
                mask = a_blk == (e_base + el)
                xm = jnp.where(mask, x_blk, jnp.bfloat16(0))
                h = jnp.maximum(
                    jnp.dot(xm, w1_ref[el],
                            preferred_element_type=jnp.float32),
                    0.0,
                )
                acc = acc + jnp.dot(
                    h.astype(jnp.bfloat16), w2_ref[el],
                    preferred_element_type=jnp.float32)
            return acc

        out_ref[...] = ffn(x_ref[...], a_ref[...])

        rdma_a.wait()
        y_rdmas = []
        for i in range(NCHUNK):
            x_rdmas[i].wait()
            sl = pl.ds(i * tc, tc)
            ys_ref[sl, :] = ffn(xr_ref[sl, :], ar_ref[sl, :]).astype(
                jnp.bfloat16)
            r = pltpu.make_async_remote_copy(
                src_ref=chunk(ys_ref, i), dst_ref=chunk(yr_ref, i),
                send_sem=y_send_sems.at[i], recv_sem=y_recv_sems.at[i],
                device_id=peer, device_id_type=pl.DeviceIdType.MESH)
            r.start()
            y_rdmas.append(r)

        for r in y_rdmas:
            r.wait()
        out_ref[...] = out_ref[...] + yr_ref[...].astype(jnp.float32)

    return pl.pallas_call(
        body,
        out_shape=jax.ShapeDtypeStruct((t, d), jnp.float32),
        in_specs=[
            pl.BlockSpec(memory_space=pltpu.VMEM),
            pl.BlockSpec(memory_space=pltpu.VMEM),
            pl.BlockSpec(memory_space=pltpu.VMEM),
            pl.BlockSpec(memory_space=pltpu.VMEM),
        ],
        out_specs=pl.BlockSpec(memory_space=pltpu.VMEM),
        scratch_shapes=[
            pltpu.VMEM((t, d), jnp.bfloat16),
            pltpu.VMEM((t, 1), jnp.int32),
            pltpu.VMEM((t, d), jnp.bfloat16),
            pltpu.VMEM((t, d), jnp.bfloat16),
            pltpu.SemaphoreType.DMA((NCHUNK,)),
            pltpu.SemaphoreType.DMA((NCHUNK,)),
            pltpu.SemaphoreType.DMA((2,)),
            pltpu.SemaphoreType.DMA((NCHUNK,)),
            pltpu.SemaphoreType.DMA((NCHUNK,)),
        ],
        compiler_params=pltpu.CompilerParams(collective_id=0),
    )(xb, assign2d, w1b, w2b)


# device time: 25849 ns/iter; 1.0801x vs baseline; 1.0801x over previous
import jax
import jax.numpy as jnp
from jax import lax
from jax.experimental import pallas as pl
from jax.experimental.pallas import tpu as pltpu


def kernel(x, assign, W1, W2):
    t, d = x.shape
    n_exp, _, f = W1.shape
    qt = t // 4
    assign2d = assign.reshape(t, 1)
    xb = x.astype(jnp.bfloat16)
    w1b = W1.astype(jnp.bfloat16)
    w2b = W2.astype(jnp.bfloat16)

    def body(x_ref, a_ref, w1_ref, w2_ref, out_ref,
             xr_ref, ar_ref, ys_ref, yr_ref, s_ref, sems):
        my_x = lax.axis_index("x")
        my_y = lax.axis_index("y")
        my_z = lax.axis_index("z")
        zpeer = (my_x, my_y, 1 - my_z)
        ynbr = (my_x, 1 - my_y, my_z)
        xnbr = (1 - my_x, my_y, my_z)
        q = 2 * my_x + my_y
        qsl = pl.ds(q * qt, qt)

        barrier = pltpu.get_barrier_semaphore()
        for nbr in (zpeer, ynbr, xnbr):
            pl.semaphore_signal(barrier, inc=1, device_id=nbr,
                                device_id_type=pl.DeviceIdType.MESH)
        pl.semaphore_wait(barrier, 3)

        rz1x = pltpu.make_async_remote_copy(
            src_ref=x_ref.at[qsl, :], dst_ref=xr_ref,
            send_sem=sems.at[0], recv_sem=sems.at[1],
            device_id=zpeer, device_id_type=pl.DeviceIdType.MESH)
        rz1x.start()
        rz1a = pltpu.make_async_remote_copy(
            src_ref=a_ref.at[qsl, :], dst_ref=ar_ref,
            send_sem=sems.at[2], recv_sem=sems.at[3],
            device_id=zpeer, device_id_type=pl.DeviceIdType.MESH)
        rz1a.start()

        e_base = 2 * my_z

        def ffn(x_blk, a_blk):
            m = x_blk.shape[0]
            acc = jnp.zeros((m, d), jnp.float32)
            for el in range(n_exp):
                mask = a_blk == (e_base + el)
                xm = jnp.where(mask, x_blk, jnp.bfloat16(0))
                h = jnp.maximum(
                    jnp.dot(xm, w1_ref[el],
                            preferred_element_type=jnp.float32), 0.0)
                acc = acc + jnp.dot(
                    h.astype(jnp.bfloat16), w2_ref[el],
                    preferred_element_type=jnp.float32)
            return acc

        s_ref[qsl, :] = ffn(x_ref[qsl, :], a_ref[qsl, :]).astype(
            jnp.bfloat16)

        rz1x.wait()
        rz1a.wait()
        ys_ref[...] = ffn(xr_ref[...], ar_ref[...]).astype(jnp.bfloat16)
        rz2 = pltpu.make_async_remote_copy(
            src_ref=ys_ref, dst_ref=yr_ref,
            send_sem=sems.at[4], recv_sem=sems.at[5],
            device_id=zpeer, device_id_type=pl.DeviceIdType.MESH)
        rz2.start()
        rz2.wait()

        s_ref[qsl, :] = s_ref[qsl, :] + yr_ref[...]

        ry = pltpu.make_async_remote_copy(
            src_ref=s_ref.at[qsl, :], dst_ref=s_ref.at[qsl, :],
            send_sem=sems.at[6], recv_sem=sems.at[7],
            device_id=ynbr, device_id_type=pl.DeviceIdType.MESH)
        ry.start()
        ry.wait()

        hsl = pl.ds(my_x * (2 * qt), 2 * qt)
        rx = pltpu.make_async_remote_copy(
            src_ref=s_ref.at[hsl, :], dst_ref=s_ref.at[hsl, :],
            send_sem=sems.at[8], recv_sem=sems.at[9],
            device_id=xnbr, device_id_type=pl.DeviceIdType.MESH)
        rx.start()
        rx.wait()

        out_ref[...] = s_ref[...].astype(jnp.float32)

    return pl.pallas_call(
        body,
        out_shape=jax.ShapeDtypeStruct((t, d), jnp.float32),
        in_specs=[
            pl.BlockSpec(memory_space=pltpu.VMEM),
            pl.BlockSpec(memory_space=pltpu.VMEM),
            pl.BlockSpec(memory_space=pltpu.VMEM),
            pl.BlockSpec(memory_space=pltpu.VMEM),
        ],
        out_specs=pl.BlockSpec(memory_space=pltpu.VMEM),
        scratch_shapes=[
            pltpu.VMEM((qt, d), jnp.bfloat16),
            pltpu.VMEM((qt, 1), jnp.int32),
            pltpu.VMEM((qt, d), jnp.bfloat16),
            pltpu.VMEM((qt, d), jnp.bfloat16),
            pltpu.VMEM((t, d), jnp.bfloat16),
            pltpu.SemaphoreType.DMA((10,)),
        ],
        compiler_params=pltpu.CompilerParams(collective_id=0),
    )(xb, assign2d, w1b, w2b)


# device time: 24531 ns/iter; 1.1382x vs baseline; 1.0537x over previous
import jax
import jax.numpy as jnp
from jax import lax
from jax.experimental import pallas as pl
from jax.experimental.pallas import tpu as pltpu


def kernel(x, assign, W1, W2):
    t, d = x.shape
    n_exp, _, f = W1.shape
    qt = t // 4
    q_out = 2 * lax.axis_index("x") + lax.axis_index("y")
    xq = lax.dynamic_slice_in_dim(x, q_out * qt, qt).astype(jnp.bfloat16)
    aq = lax.dynamic_slice_in_dim(assign, q_out * qt, qt).reshape(qt, 1)
    w1b = W1.astype(jnp.bfloat16)
    w2b = W2.astype(jnp.bfloat16)

    def body(x_ref, a_ref, w1_ref, w2_ref, out_ref,
             xr_ref, ar_ref, ys_ref, yr_ref, s_ref, sems):
        my_x = lax.axis_index("x")
        my_y = lax.axis_index("y")
        my_z = lax.axis_index("z")
        zpeer = (my_x, my_y, 1 - my_z)
        ynbr = (my_x, 1 - my_y, my_z)
        xnbr = (1 - my_x, my_y, my_z)
        q = 2 * my_x + my_y
        qsl = pl.ds(q * qt, qt)

        barrier = pltpu.get_barrier_semaphore()
        for nbr in (zpeer, ynbr, xnbr):
            pl.semaphore_signal(barrier, inc=1, device_id=nbr,
                                device_id_type=pl.DeviceIdType.MESH)
        pl.semaphore_wait(barrier, 3)

        rz1x = pltpu.make_async_remote_copy(
            src_ref=x_ref, dst_ref=xr_ref,
            send_sem=sems.at[0], recv_sem=sems.at[1],
            device_id=zpeer, device_id_type=pl.DeviceIdType.MESH)
        rz1x.start()
        rz1a = pltpu.make_async_remote_copy(
            src_ref=a_ref, dst_ref=ar_ref,
            send_sem=sems.at[2], recv_sem=sems.at[3],
            device_id=zpeer, device_id_type=pl.DeviceIdType.MESH)
        rz1a.start()

        e_base = 2 * my_z

        def ffn(x_blk, a_blk):
            m = x_blk.shape[0]
            acc = jnp.zeros((m, d), jnp.float32)
            for el in range(n_exp):
                mask = a_blk == (e_base + el)
                xm = jnp.where(mask, x_blk, jnp.bfloat16(0))
                h = jnp.maximum(
                    jnp.dot(xm, w1_ref[el],
                            preferred_element_type=jnp.float32), 0.0)
                acc = acc + jnp.dot(
                    h.astype(jnp.bfloat16), w2_ref[el],
                    preferred_element_type=jnp.float32)
            return acc

        s_ref[qsl, :] = ffn(x_ref[...], a_ref[...]).astype(jnp.bfloat16)

        rz1x.wait()
        rz1a.wait()
        ys_ref[...] = ffn(xr_ref[...], ar_ref[...]).astype(jnp.bfloat16)
        rz2 = pltpu.make_async_remote_copy(
            src_ref=ys_ref, dst_ref=yr_ref,
            send_sem=sems.at[4], recv_sem=sems.at[5],
            device_id=zpeer, device_id_type=pl.DeviceIdType.MESH)
        rz2.start()
        rz2.wait()

        s_ref[qsl, :] = s_ref[qsl, :] + yr_ref[...]

        ry = pltpu.make_async_remote_copy(
            src_ref=s_ref.at[qsl, :], dst_ref=s_ref.at[qsl, :],
            send_sem=sems.at[6], recv_sem=sems.at[7],
            device_id=ynbr, device_id_type=pl.DeviceIdType.MESH)
        ry.start()
        ry.wait()

        hsl = pl.ds(my_x * (2 * qt), 2 * qt)
        rx = pltpu.make_async_remote_copy(
            src_ref=s_ref.at[hsl, :], dst_ref=s_ref.at[hsl, :],
            send_sem=sems.at[8], recv_sem=sems.at[9],
            device_id=xnbr, device_id_type=pl.DeviceIdType.MESH)
        rx.start()
        rx.wait()

        out_ref[...] = s_ref[...].astype(jnp.float32)

    return pl.pallas_call(
        body,
        out_shape=jax.ShapeDtypeStruct((t, d), jnp.float32),
        in_specs=[
            pl.BlockSpec(memory_space=pltpu.VMEM),
            pl.BlockSpec(memory_space=pltpu.VMEM),
            pl.BlockSpec(memory_space=pltpu.VMEM),
            pl.BlockSpec(memory_space=pltpu.VMEM),
        ],
        out_specs=pl.BlockSpec(memory_space=pltpu.VMEM),
        scratch_shapes=[
            pltpu.VMEM((qt, d), jnp.bfloat16),
            pltpu.VMEM((qt, 1), jnp.int32),
            pltpu.VMEM((qt, d), jnp.bfloat16),
            pltpu.VMEM((qt, d), jnp.bfloat16),
            pltpu.VMEM((t, d), jnp.bfloat16),
            pltpu.SemaphoreType.DMA((10,)),
        ],
        compiler_params=pltpu.CompilerParams(collective_id=0),
    )(xq, aq, w1b, w2b)


# device time: 21982 ns/iter; 1.2701x vs baseline; 1.1160x over previous
import jax
import jax.numpy as jnp
from jax import lax
from jax.experimental import pallas as pl
from jax.experimental.pallas import tpu as pltpu


def kernel(x, assign, W1, W2):
    t, d = x.shape
    n_exp, _, f = W1.shape
    qt = t // 4
    q_out = 2 * lax.axis_index("x") + lax.axis_index("y")
    xq = lax.dynamic_slice_in_dim(x, q_out * qt, qt).astype(jnp.bfloat16)
    aq = lax.dynamic_slice_in_dim(assign, q_out * qt, qt).reshape(qt, 1)
    w1b = W1.astype(jnp.bfloat16)
    w2b = W2.astype(jnp.bfloat16)

    def body(x_ref, a_ref, w1_ref, w2_ref, out_ref,
             xr_ref, ar_ref, ys_ref, yr_ref, s_ref, sems):
        my_x = lax.axis_index("x")
        my_y = lax.axis_index("y")
        my_z = lax.axis_index("z")
        zpeer = (my_x, my_y, 1 - my_z)
        ynbr = (my_x, 1 - my_y, my_z)
        xnbr = (1 - my_x, my_y, my_z)
        diag = (1 - my_x, 1 - my_y, my_z)
        q = 2 * my_x + my_y
        qsl = pl.ds(q * qt, qt)

        barrier = pltpu.get_barrier_semaphore()
        for nbr in (zpeer, ynbr, xnbr, diag):
            pl.semaphore_signal(barrier, inc=1, device_id=nbr,
                                device_id_type=pl.DeviceIdType.MESH)
        pl.semaphore_wait(barrier, 4)

        rz1x = pltpu.make_async_remote_copy(
            src_ref=x_ref, dst_ref=xr_ref,
            send_sem=sems.at[0], recv_sem=sems.at[1],
            device_id=zpeer, device_id_type=pl.DeviceIdType.MESH)
        rz1x.start()
        rz1a = pltpu.make_async_remote_copy(
            src_ref=a_ref, dst_ref=ar_ref,
            send_sem=sems.at[2], recv_sem=sems.at[3],
            device_id=zpeer, device_id_type=pl.DeviceIdType.MESH)
        rz1a.start()

        e_base = 2 * my_z

        def ffn(x_blk, a_blk):
            m = x_blk.shape[0]
            acc = jnp.zeros((m, d), jnp.float32)
            for el in range(n_exp):
                mask = a_blk == (e_base + el)
                xm = jnp.where(mask, x_blk, jnp.bfloat16(0))
                h = jnp.maximum(
                    jnp.dot(xm, w1_ref[el],
                            preferred_element_type=jnp.float32), 0.0)
                acc = acc + jnp.dot(
                    h.astype(jnp.bfloat16), w2_ref[el],
                    preferred_element_type=jnp.float32)
            return acc

        s_ref[qsl, :] = ffn(x_ref[...], a_ref[...]).astype(jnp.bfloat16)

        rz1x.wait()
        rz1a.wait()
        ys_ref[...] = ffn(xr_ref[...], ar_ref[...]).astype(jnp.bfloat16)
        rz2 = pltpu.make_async_remote_copy(
            src_ref=ys_ref, dst_ref=yr_ref,
            send_sem=sems.at[4], recv_sem=sems.at[5],
            device_id=zpeer, device_id_type=pl.DeviceIdType.MESH)
        rz2.start()
        rz2.wait()

        s_ref[qsl, :] = s_ref[qsl, :] + yr_ref[...]

        swaps = []
        for k, nbr in enumerate((ynbr, xnbr, diag)):
            r = pltpu.make_async_remote_copy(
                src_ref=s_ref.at[qsl, :], dst_ref=s_ref.at[qsl, :],
                send_sem=sems.at[6 + 2 * k], recv_sem=sems.at[7 + 2 * k],
                device_id=nbr, device_id_type=pl.DeviceIdType.MESH)
            r.start()
            swaps.append(r)
        for r in swaps:
            r.wait()

        out_ref[...] = s_ref[...].astype(jnp.float32)

    return pl.pallas_call(
        body,
        out_shape=jax.ShapeDtypeStruct((t, d), jnp.float32),
        in_specs=[
            pl.BlockSpec(memory_space=pltpu.VMEM),
            pl.BlockSpec(memory_space=pltpu.VMEM),
            pl.BlockSpec(memory_space=pltpu.VMEM),
            pl.BlockSpec(memory_space=pltpu.VMEM),
        ],
        out_specs=pl.BlockSpec(memory_space=pltpu.VMEM),
        scratch_shapes=[
            pltpu.VMEM((qt, d), jnp.bfloat16),
            pltpu.VMEM((qt, 1), jnp.int32),
            pltpu.VMEM((qt, d), jnp.bfloat16),
            pltpu.VMEM((qt, d), jnp.bfloat16),
            pltpu.VMEM((t, d), jnp.bfloat16),
            pltpu.SemaphoreType.DMA((12,)),
        ],
        compiler_params=pltpu.CompilerParams(collective_id=0),
    )(xq, aq, w1b, w2b)


# device time: 21353 ns/iter; 1.3075x vs baseline; 1.0295x over previous
import jax
import jax.numpy as jnp
from jax import lax
from jax.experimental import pallas as pl
from jax.experimental.pallas import tpu as pltpu


def kernel(x, assign, W1, W2):
    t, d = x.shape
    n_exp, _, f = W1.shape
    qt = t // 4
    q_out = 2 * lax.axis_index("x") + lax.axis_index("y")
    xq = lax.dynamic_slice_in_dim(x, q_out * qt, qt).astype(jnp.bfloat16)
    aq = lax.dynamic_slice_in_dim(assign, q_out * qt, qt)
    w1b = W1.astype(jnp.bfloat16)
    w2b = W2.astype(jnp.bfloat16)

    def body(x_ref, a_ref, w1_ref, w2_ref, out_ref,
             xr_ref, ar_ref, ys_ref, yr_ref, s_ref, sf_ref, sems):
        my_x = lax.axis_index("x")
        my_y = lax.axis_index("y")
        my_z = lax.axis_index("z")
        zpeer = (my_x, my_y, 1 - my_z)
        ynbr = (my_x, 1 - my_y, my_z)
        xnbr = (1 - my_x, my_y, my_z)
        diag = (1 - my_x, 1 - my_y, my_z)
        q = 2 * my_x + my_y
        qsl = pl.ds(q * qt, qt)

        barrier = pltpu.get_barrier_semaphore()
        for nbr in (zpeer, ynbr, xnbr, diag):
            pl.semaphore_signal(barrier, inc=1, device_id=nbr,
                                device_id_type=pl.DeviceIdType.MESH)
        pl.semaphore_wait(barrier, 4)

        rz1x = pltpu.make_async_remote_copy(
            src_ref=x_ref, dst_ref=xr_ref,
            send_sem=sems.at[0], recv_sem=sems.at[1],
            device_id=zpeer, device_id_type=pl.DeviceIdType.MESH)
        rz1x.start()
        rz1a = pltpu.make_async_remote_copy(
            src_ref=a_ref, dst_ref=ar_ref,
            send_sem=sems.at[2], recv_sem=sems.at[3],
            device_id=zpeer, device_id_type=pl.DeviceIdType.MESH)
        rz1a.start()

        e_base = 2 * my_z

        def ffn(x_blk, a_blk):
            m = x_blk.shape[0]
            a_col = a_blk.reshape(m, 1)
            acc = jnp.zeros((m, d), jnp.float32)
            for el in range(n_exp):
                mask = a_col == (e_base + el)
                xm = jnp.where(mask, x_blk, jnp.bfloat16(0))
                h = jnp.maximum(
                    jnp.dot(xm, w1_ref[el],
                            preferred_element_type=jnp.float32), 0.0)
                acc = acc + jnp.dot(
                    h.astype(jnp.bfloat16), w2_ref[el],
                    preferred_element_type=jnp.float32)
            return acc

        s_ref[qsl, :] = ffn(x_ref[...], a_ref[...]).astype(jnp.bfloat16)

        rz1x.wait()
        rz1a.wait()
        ys_ref[...] = ffn(xr_ref[...], ar_ref[...]).astype(jnp.bfloat16)
        rz2 = pltpu.make_async_remote_copy(
            src_ref=ys_ref, dst_ref=yr_ref,
            send_sem=sems.at[4], recv_sem=sems.at[5],
            device_id=zpeer, device_id_type=pl.DeviceIdType.MESH)
        rz2.start()
        rz2.wait()

        s_ref[qsl, :] = s_ref[qsl, :] + yr_ref[...]

        swaps = []
        for k, nbr in enumerate((ynbr, xnbr, diag)):
            r = pltpu.make_async_remote_copy(
                src_ref=s_ref.at[qsl, :], dst_ref=s_ref.at[qsl, :],
                send_sem=sems.at[6 + 2 * k], recv_sem=sems.at[7 + 2 * k],
                device_id=nbr, device_id_type=pl.DeviceIdType.MESH)
            r.start()
            swaps.append(r)
        for r in swaps:
            r.wait()

        sf_ref[...] = s_ref[...].astype(jnp.float32)
        st = pltpu.make_async_copy(sf_ref, out_ref, sems.at[12])
        st.start()
        st.wait()

    return pl.pallas_call(
        body,
        out_shape=jax.ShapeDtypeStruct((t, d), jnp.float32),
        in_specs=[
            pl.BlockSpec(memory_space=pltpu.VMEM),
            pl.BlockSpec(memory_space=pltpu.VMEM),
            pl.BlockSpec(memory_space=pltpu.VMEM),
            pl.BlockSpec(memory_space=pltpu.VMEM),
        ],
        out_specs=pl.BlockSpec(memory_space=pltpu.MemorySpace.HBM),
        scratch_shapes=[
            pltpu.VMEM((qt, d), jnp.bfloat16),
            pltpu.VMEM((qt,), jnp.int32),
            pltpu.VMEM((qt, d), jnp.bfloat16),
            pltpu.VMEM((qt, d), jnp.bfloat16),
            pltpu.VMEM((t, d), jnp.bfloat16),
            pltpu.VMEM((t, d), jnp.float32),
            pltpu.SemaphoreType.DMA((13,)),
        ],
        compiler_params=pltpu.CompilerParams(collective_id=0),
    )(xq, aq, w1b, w2b)
